# baseline (speedup 1.0000x reference)
import functools

import jax
import jax.numpy as jnp
from jax import lax
from jax.experimental import pallas as pl
from jax.experimental.pallas import tpu as pltpu
from jax.experimental.pallas import tpu_sc as plsc
from jax.experimental.layout import Layout, with_layout_constraint

B, SEQ, EMBED = 4096, 200, 32
VOCAB = 1000000
VPAD = 1000064
NC, NS, L = 2, 16, 16
NW = NC * NS

SEQ_PER_W = B // NW
SPB = 8
NBLK = SEQ_PER_W // SPB
NT = NBLK // 2
IDXP = 216
SPLITS = ((0, 104), (104, 96))


def _fire_gathers(table_hbm, idxb, rowsb, gsem):
    for s in range(SPB):
        for o, n in SPLITS:
            pltpu.async_copy(
                table_hbm.at[idxb.at[s, pl.ds(o, n)]],
                rowsb.at[s, pl.ds(o, n)], gsem)


def _drain(src, dst, sem):
    pltpu.make_async_copy(src, dst, sem).wait()


def _fix_zero_rows(idxb, rowsb):
    offs = [i * L for i in range(SEQ // L)] + [SEQ - L]
    m_acc = idxb[0, pl.ds(0, L)] == jnp.int32(0)
    first = True
    for s in range(SPB):
        for o in offs:
            if first:
                first = False
                continue
            m_acc = m_acc | (idxb[s, pl.ds(o, L)] == jnp.int32(0))
    mi = jnp.where(m_acc, jnp.int32(1), jnp.int32(0))
    dnums = lax.GatherDimensionNumbers(
        offset_dims=(), collapsed_slice_dims=(0,), start_index_map=(0,))
    for k in (1, 2, 4, 8):
        perm = (lax.iota(jnp.int32, L) ^ jnp.int32(k)).reshape(L, 1)
        mi = mi | lax.gather(mi, perm, dnums, slice_sizes=(1,),
                             mode=lax.GatherScatterMode.PROMISE_IN_BOUNDS)

    @pl.when(mi[0] > 0)
    def _fix():
        def fix_row(r, c):
            s = r // SEQ
            rr = r % SEQ
            v = idxb[s, pl.ds(rr, L)][0]

            @pl.when(v == jnp.int32(0))
            def _zero():
                z = jnp.zeros((L,), jnp.float32)
                rowsb[s, rr, pl.ds(0, L)] = z
                rowsb[s, rr, pl.ds(L, L)] = z
            return c
        lax.fori_loop(0, SPB * SEQ, fix_row, 0)


def _embed_body(seq_hbm, table_hbm, out_hbm,
                idx0, idx1, rows0, rows1, gsem, isem, ssem):
    wid = lax.axis_index("s") * NC + lax.axis_index("c")
    wseq = wid * SEQ_PER_W

    def idx_src(b):
        return seq_hbm.at[pl.ds(wseq + b * SPB, SPB)]

    def out_dst(b):
        return out_hbm.at[pl.ds(wseq + b * SPB, SPB)]

    def idx_dst(buf):
        return buf.at[pl.ds(0, SPB), pl.ds(0, SEQ)]

    pltpu.sync_copy(idx_src(0), idx_dst(idx0))
    pltpu.async_copy(idx_src(1), idx_dst(idx1), isem)
    _fire_gathers(table_hbm, idx0, rows0, gsem)

    def step(t, carry):
        a = 2 * t
        b = a + 1
        not_last = t < NT - 1

        _drain(idx_src(0), idx_dst(idx1), isem)

        @pl.when(t > 0)
        def _():
            _drain(rows1, out_dst(0), ssem)
        _fire_gathers(table_hbm, idx1, rows1, gsem)

        _drain(out_dst(0), rows0, gsem)
        _fix_zero_rows(idx0, rows0)

        @pl.when(not_last)
        def _():
            pltpu.async_copy(idx_src(a + 2), idx_dst(idx0), isem)
        pltpu.async_copy(rows0, out_dst(a), ssem)

        @pl.when(not_last)
        def _():
            _drain(idx_src(0), idx_dst(idx0), isem)
        _drain(rows0, out_dst(0), ssem)

        @pl.when(not_last)
        def _():
            _fire_gathers(table_hbm, idx0, rows0, gsem)

        _drain(out_dst(0), rows1, gsem)
        _fix_zero_rows(idx1, rows1)

        @pl.when(not_last)
        def _():
            pltpu.async_copy(idx_src(b + 2), idx_dst(idx1), isem)
        pltpu.async_copy(rows1, out_dst(b), ssem)
        return carry

    lax.fori_loop(0, NT, step, 0)
    _drain(rows1, out_dst(0), ssem)


def _gather_call(full_seq, table_lin):
    mesh = plsc.VectorSubcoreMesh(core_axis_name="c", subcore_axis_name="s")
    fn = functools.partial(
        pl.kernel,
        mesh=mesh,
        compiler_params=pltpu.CompilerParams(use_tc_tiling_on_sc=False),
        out_type=jax.ShapeDtypeStruct((B, SEQ, EMBED), jnp.float32),
        scratch_types=[
            pltpu.VMEM((SPB, IDXP), jnp.int32),
            pltpu.VMEM((SPB, IDXP), jnp.int32),
            pltpu.VMEM((SPB, SEQ, EMBED), jnp.float32),
            pltpu.VMEM((SPB, SEQ, EMBED), jnp.float32),
            pltpu.SemaphoreType.DMA,
            pltpu.SemaphoreType.DMA,
            pltpu.SemaphoreType.DMA,
        ],
    )(_embed_body)
    return fn(full_seq, table_lin)


def _impl(full_seq, table):
    out = _gather_call(full_seq, table)
    return with_layout_constraint(
        out, Layout(major_to_minor=(0, 1, 2), tiling=((8,),)))


def kernel(full_seq, table):
    return _impl(full_seq, table)
